# Initial kernel scaffold; baseline (speedup 1.0000x reference)
import jax
import jax.numpy as jnp
from jax.experimental import pallas as pl


def kernel(logits_trg, x_trg_2, x_ema_2, img_trg):
    raise NotImplementedError("write your pallas kernel here")


# speedup vs baseline: 5.6842x; 5.6842x over previous
import jax
import jax.numpy as jnp
from jax.experimental import pallas as pl
from jax.experimental.pallas import tpu as pltpu

_N = 1024
_PAD = 128
_CH = 256
_CLS = 19
_TOPK = 4
_INV_SIGMA2 = 1.0 / (24.0 * 24.0)
_W0, _W1 = 1.0, 0.5
_PAR_DELTA = ((-1, 0, 0), (0, 0, 1))


def _wrap_mask(t_lane, dc):
    if dc == -1:
        return t_lane > 0
    if dc == 1:
        return t_lane < 31
    return None


def _tc_body(lq_ref, f_ref, dens_ref, pos_ref, neg_ref, cnt_ref,
             ppad_ref, spad_ref, fpad_ref):
    npad = _N + 2 * _PAD
    ppad_ref[...] = jnp.zeros((8, _CLS, npad), jnp.float32)
    spad_ref[...] = jnp.zeros((8, npad), jnp.float32)
    fpad_ref[...] = jnp.zeros((2, _CH, npad), jnp.float32)
    for r in range(8):
        x = lq_ref[r]
        m = jnp.max(x, axis=0, keepdims=True)
        e = jnp.exp(x - m)
        p = e / jnp.sum(e, axis=0, keepdims=True)
        ppad_ref[r, :, _PAD:_PAD + _N] = p
        spad_ref[r, _PAD:_PAD + _N] = jnp.sum(p, axis=0)
    fpad_ref[:, :, _PAD:_PAD + _N] = f_ref[...]

    t_lane = jax.lax.broadcasted_iota(jnp.int32, (1, _N), 1) % 32

    sim32 = {}
    for dr in (-1, 0, 1):
        for dc in (-1, 0, 1):
            o = 32 * dr + dc
            wm = _wrap_mask(t_lane, dc)
            d = jnp.zeros((2, _N), jnp.float32)
            for c0 in range(0, _CH, 32):
                fc = f_ref[:, c0:c0 + 32, :]
                fsh = fpad_ref[:, c0:c0 + 32, _PAD + o:_PAD + o + _N]
                if wm is not None:
                    fsh = jnp.where(wm[:, None, :], fsh, 0.0)
                d = d + jnp.sum((fsh - fc) ** 2, axis=1)
            sim32[(dr, dc)] = jnp.exp(d * (-_INV_SIGMA2))

    f0 = f_ref[:, 0, :]
    maskf = (f0 > 0.0).astype(jnp.float32)

    pos_acc = jnp.float32(0.0)
    neg_acc = jnp.float32(0.0)
    for ph in (0, 1):
        for pw in (0, 1):
            q = ph * 2 + pw
            pq = ppad_ref[2 * q:2 * q + 2, :, _PAD:_PAD + _N]
            sq = spad_ref[2 * q:2 * q + 2, _PAD:_PAD + _N]
            sims, cposs, cnegs = [], [], []
            for i in range(3):
                for j in range(3):
                    dr = _PAR_DELTA[ph][i]
                    dc = _PAR_DELTA[pw][j]
                    o = 32 * dr + dc
                    q2 = ((ph + i + 1) % 2) * 2 + ((pw + j + 1) % 2)
                    psh = ppad_ref[2 * q2:2 * q2 + 2, :, _PAD + o:_PAD + o + _N]
                    ssh = spad_ref[2 * q2:2 * q2 + 2, _PAD + o:_PAD + o + _N]
                    wm = _wrap_mask(t_lane, dc)
                    if wm is not None:
                        psh = jnp.where(wm[:, None, :], psh, 0.0)
                        ssh = jnp.where(wm, ssh, 0.0)
                    cp = jnp.sum(pq * psh, axis=1)
                    cn = sq * ssh - cp
                    sims.append(sim32[(dr, dc)])
                    cposs.append(cp)
                    cnegs.append(cn)
            dtot = sims[0]
            for k in range(1, 9):
                dtot = dtot + sims[k]
            dens_ref[2 * q:2 * q + 2, :] = 1.0 - dtot * (1.0 / 9.0)
            qacc_p = jnp.zeros((2, _N), jnp.float32)
            qacc_n = jnp.zeros((2, _N), jnp.float32)
            for k in range(9):
                rmax = jnp.zeros((2, _N), jnp.int32)
                rmin = jnp.zeros((2, _N), jnp.int32)
                for k2 in range(9):
                    if k2 == k:
                        continue
                    gt = sims[k2] > sims[k]
                    lt = sims[k2] < sims[k]
                    if k2 < k:
                        eqv = sims[k2] == sims[k]
                        gt = gt | eqv
                        lt = lt | eqv
                    rmax = rmax + gt.astype(jnp.int32)
                    rmin = rmin + lt.astype(jnp.int32)
                qacc_p = qacc_p + jnp.where(rmax < _TOPK + 1,
                                            sims[k] * (-cposs[k]), 0.0)
                qacc_n = qacc_n + jnp.where(rmin < _TOPK,
                                            (1.0 - sims[k]) * (-cnegs[k]), 0.0)
            pos_acc = pos_acc + jnp.sum(qacc_p * maskf)
            neg_acc = neg_acc + jnp.sum(qacc_n * maskf)

    pos_ref[...] = jnp.full((1, 1), pos_acc, jnp.float32)
    neg_ref[...] = jnp.full((1, 1), neg_acc, jnp.float32)
    cnt_ref[...] = jnp.full((1, 1), jnp.sum(maskf), jnp.float32)


def _tc_call(lq, feats):
    npad = _N + 2 * _PAD
    return pl.pallas_call(
        _tc_body,
        out_shape=[
            jax.ShapeDtypeStruct((8, _N), jnp.float32),
            jax.ShapeDtypeStruct((1, 1), jnp.float32),
            jax.ShapeDtypeStruct((1, 1), jnp.float32),
            jax.ShapeDtypeStruct((1, 1), jnp.float32),
        ],
        scratch_shapes=[
            pltpu.VMEM((8, _CLS, npad), jnp.float32),
            pltpu.VMEM((8, npad), jnp.float32),
            pltpu.VMEM((2, _CH, npad), jnp.float32),
        ],
    )(lq, feats)


def kernel(logits_trg, x_trg_2, x_ema_2, img_trg):
    B, C, H, W = logits_trg.shape
    del x_trg_2, img_trg
    lq = jnp.stack([logits_trg[:, :, ph::2, pw::2]
                    for ph in (0, 1) for pw in (0, 1)])
    lq = lq.reshape(4 * B, C, _N)
    feats = x_ema_2.reshape(B, _CH, _N)
    dens, ps, ns, cnt32 = _tc_call(lq, feats)
    cnt = cnt32[0, 0] * 4.0
    loss_pos = ps[0, 0] / (cnt * (_TOPK + 1)) * _W0
    loss_neg = ns[0, 0] / (cnt * _TOPK) * _W1
    density = (dens.reshape(2, 2, B, 32, 32)
               .transpose(2, 3, 0, 4, 1)
               .reshape(B, 1, H, W))
    return loss_pos, loss_neg, density
